# baseline (device time: 233462 ns/iter reference)
import jax
import jax.numpy as jnp
from jax import lax
from jax.experimental import pallas as pl
from jax.experimental.pallas import tpu as pltpu

M, N = 8192, 1024
HALF = M // 2
QTR = HALF // 2
K = 16
R = QTR // K
KH = K // 2

MESH = pl.DeviceIdType.MESH


def kernel(x):
    def body(x_hbm, out_hbm, vxq, vrx, vzy, vyr,
             xs, xr, zns, znr, yns, ynr, yfs, yfr, zfs, zfr,
             ld_sems, st_sems):
        mx = lax.axis_index("x")
        my = lax.axis_index("y")
        mz = lax.axis_index("z")
        p = mz % 2
        xp = (1 - mx, my, mz)
        yp = (mx, 1 - my, mz)
        zp = (mx, my, mz + 1 - 2 * p)

        barrier_sem = pltpu.get_barrier_semaphore()
        for nbr in (xp, yp, zp):
            pl.semaphore_signal(barrier_sem, inc=1,
                                device_id=nbr, device_id_type=MESH)
        pl.semaphore_wait(barrier_sem, 3)

        xrow0 = my * HALF + p * QTR
        zrow0 = my * HALF + (1 - p) * QTR
        yrow0 = (1 - my) * HALF + p * QTR
        diag0 = (1 - my) * HALF + (1 - p) * QTR

        lds = []
        for c in range(K):
            ld = pltpu.make_async_copy(
                x_hbm.at[pl.ds(xrow0 + c * R, R)], vxq.at[c],
                ld_sems.at[c])
            ld.start()
            lds.append(ld)

        x_rdmas = []
        for c in range(K):
            lds[c].wait()
            r = pltpu.make_async_remote_copy(
                src_ref=vxq.at[c], dst_ref=vrx.at[c],
                send_sem=xs.at[c], recv_sem=xr.at[c],
                device_id=xp, device_id_type=MESH)
            r.start()
            x_rdmas.append(r)

        zn_recv = [
            pltpu.make_async_remote_copy(
                src_ref=vzy.at[c], dst_ref=vzy.at[c],
                send_sem=zns.at[c], recv_sem=znr.at[c],
                device_id=zp, device_id_type=MESH)
            for c in range(K)
        ]
        yn_recv = [
            pltpu.make_async_remote_copy(
                src_ref=vyr.at[c], dst_ref=vyr.at[c],
                send_sem=yns.at[c], recv_sem=ynr.at[c],
                device_id=yp, device_id_type=MESH)
            for c in range(K)
        ]
        yf_recv = [
            pltpu.make_async_remote_copy(
                src_ref=vzy.at[c],
                dst_ref=out_hbm.at[pl.ds(diag0 + c * R, R)],
                send_sem=yfs.at[c], recv_sem=yfr.at[c],
                device_id=yp, device_id_type=MESH)
            for c in range(KH)
        ]
        zf_recv = [
            pltpu.make_async_remote_copy(
                src_ref=vyr.at[KH + c],
                dst_ref=out_hbm.at[pl.ds(diag0 + (KH + c) * R, R)],
                send_sem=zfs.at[c], recv_sem=zfr.at[c],
                device_id=zp, device_id_type=MESH)
            for c in range(KH)
        ]

        sends = []
        sts = []
        st_i = 0
        for c in range(K):
            x_rdmas[c].wait_recv()
            vrx[c] = vrx[c] + vxq[c]
            zn = pltpu.make_async_remote_copy(
                src_ref=vrx.at[c], dst_ref=vzy.at[c],
                send_sem=zns.at[c], recv_sem=znr.at[c],
                device_id=zp, device_id_type=MESH)
            zn.start()
            sends.append(zn)
            yn = pltpu.make_async_remote_copy(
                src_ref=vrx.at[c], dst_ref=vyr.at[c],
                send_sem=yns.at[c], recv_sem=ynr.at[c],
                device_id=yp, device_id_type=MESH)
            yn.start()
            sends.append(yn)
            st = pltpu.make_async_copy(
                vrx.at[c], out_hbm.at[pl.ds(xrow0 + c * R, R)],
                st_sems.at[st_i])
            st.start()
            sts.append(st)
            st_i += 1

            zn_recv[c].wait_recv()
            st = pltpu.make_async_copy(
                vzy.at[c], out_hbm.at[pl.ds(zrow0 + c * R, R)],
                st_sems.at[st_i])
            st.start()
            sts.append(st)
            st_i += 1
            if c < KH:
                yf = pltpu.make_async_remote_copy(
                    src_ref=vzy.at[c],
                    dst_ref=out_hbm.at[pl.ds(zrow0 + c * R, R)],
                    send_sem=yfs.at[c], recv_sem=yfr.at[c],
                    device_id=yp, device_id_type=MESH)
                yf.start()
                sends.append(yf)

            yn_recv[c].wait_recv()
            st = pltpu.make_async_copy(
                vyr.at[c], out_hbm.at[pl.ds(yrow0 + c * R, R)],
                st_sems.at[st_i])
            st.start()
            sts.append(st)
            st_i += 1
            if c >= KH:
                zf = pltpu.make_async_remote_copy(
                    src_ref=vyr.at[c],
                    dst_ref=out_hbm.at[pl.ds(yrow0 + c * R, R)],
                    send_sem=zfs.at[c - KH], recv_sem=zfr.at[c - KH],
                    device_id=zp, device_id_type=MESH)
                zf.start()
                sends.append(zf)

        for c in range(KH):
            yf_recv[c].wait_recv()
            zf_recv[c].wait_recv()

        for r in x_rdmas:
            r.wait_send()
        for s in sends:
            s.wait_send()
        for s in sts:
            s.wait()

    return pl.pallas_call(
        body,
        out_shape=jax.ShapeDtypeStruct((M, N), jnp.float32),
        in_specs=[pl.BlockSpec(memory_space=pl.ANY)],
        out_specs=pl.BlockSpec(memory_space=pl.ANY),
        scratch_shapes=[
            pltpu.VMEM((K, R, N), jnp.float32),
            pltpu.VMEM((K, R, N), jnp.float32),
            pltpu.VMEM((K, R, N), jnp.float32),
            pltpu.VMEM((K, R, N), jnp.float32),
            pltpu.SemaphoreType.DMA((K,)),
            pltpu.SemaphoreType.DMA((K,)),
            pltpu.SemaphoreType.DMA((K,)),
            pltpu.SemaphoreType.DMA((K,)),
            pltpu.SemaphoreType.DMA((K,)),
            pltpu.SemaphoreType.DMA((K,)),
            pltpu.SemaphoreType.DMA((KH,)),
            pltpu.SemaphoreType.DMA((KH,)),
            pltpu.SemaphoreType.DMA((KH,)),
            pltpu.SemaphoreType.DMA((KH,)),
            pltpu.SemaphoreType.DMA((K,)),
            pltpu.SemaphoreType.DMA((3 * K,)),
        ],
        compiler_params=pltpu.CompilerParams(collective_id=0),
    )(x)
